# baseline (speedup 1.0000x reference)
import functools

import jax
import jax.numpy as jnp
from jax import lax
from jax.experimental import pallas as pl
from jax.experimental.pallas import tpu as pltpu
from jax.experimental.pallas import tpu_sc as plsc

_NUM_CORES = 2
_NUM_SUBCORES = 16
_NUM_WORKERS = _NUM_CORES * _NUM_SUBCORES
_CHUNK = 128
_LANES = 16


@functools.lru_cache(maxsize=None)
def _make_gather(n_chunks: int, n_seq: int, n_bblk: int):
    dim = 64
    mesh = plsc.VectorSubcoreMesh(core_axis_name="c", subcore_axis_name="s")

    @functools.partial(
        pl.kernel,
        mesh=mesh,
        out_type=jax.ShapeDtypeStruct(
            (n_seq, dim // 8, n_bblk, 8, _CHUNK), jnp.float32
        ),
        scratch_types=[
            pltpu.VMEM((n_chunks, _CHUNK), jnp.int32),
            pltpu.VMEM((_CHUNK, dim), jnp.float32),
            pltpu.VMEM((_CHUNK, dim), jnp.float32),
            pltpu.VMEM((dim // 8, 8, _CHUNK), jnp.float32),
            pltpu.VMEM((dim // 8, 8, _CHUNK), jnp.float32),
            pltpu.SemaphoreType.DMA,
            pltpu.SemaphoreType.DMA,
            pltpu.SemaphoreType.DMA,
            pltpu.SemaphoreType.DMA,
        ],
        compiler_params=pltpu.CompilerParams(
            use_tc_tiling_on_sc=False, needs_layout_passes=False
        ),
    )
    def gather_kernel(table_hbm, idx_hbm, out_hbm, idx_v, gb0, gb1, tb0, tb1,
                      gs0, gs1, ss0, ss1):
        wid = lax.axis_index("s") * _NUM_CORES + lax.axis_index("c")
        pltpu.sync_copy(idx_hbm.at[wid], idx_v)
        gbufs = (gb0, gb1)
        tbufs = (tb0, tb1)
        gsems = (gs0, gs1)
        ssems = (ss0, ss1)
        lanes = lax.iota(jnp.int32, _LANES)

        def fire_gather(j, p):
            pltpu.async_copy(table_hbm.at[idx_v.at[j]], gbufs[p], gsems[p])

        def drain_gather(p):
            pltpu.make_async_copy(
                table_hbm.at[pl.ds(0, _CHUNK)], gbufs[p], gsems[p]
            ).wait()

        def transpose(p):
            gb = gbufs[p]
            tb = tbufs[p]

            def body(d, carry):
                cols = jnp.full((_LANES,), d, jnp.int32)
                tr = d // 8
                dr = lax.rem(d, 8)
                for g in range(_CHUNK // _LANES):
                    rows = g * _LANES + lanes
                    v = plsc.load_gather(gb, [rows, cols])
                    tb[tr, dr, pl.ds(g * _LANES, _LANES)] = v
                return carry

            lax.fori_loop(0, dim, body, 0)

        def fire_store(j, p):
            c = wid * n_chunks + j
            s = c // n_bblk
            bblk = lax.rem(c, n_bblk)
            pltpu.async_copy(tbufs[p], out_hbm.at[s].at[:, bblk], ssems[p])

        def wait_store(p):
            pltpu.make_async_copy(
                tbufs[p], out_hbm.at[0].at[:, 0], ssems[p]
            ).wait()

        fire_gather(0, 0)
        fire_gather(1, 1)
        drain_gather(0)
        transpose(0)
        fire_store(0, 0)
        fire_gather(2, 0)
        drain_gather(1)
        transpose(1)
        fire_store(1, 1)
        fire_gather(3, 1)

        def pair(t, carry):
            j = 2 * t + 2
            drain_gather(0)
            wait_store(0)
            transpose(0)
            fire_store(j, 0)
            fire_gather(j + 2, 0)
            drain_gather(1)
            wait_store(1)
            transpose(1)
            fire_store(j + 1, 1)
            fire_gather(j + 3, 1)
            return carry

        lax.fori_loop(0, (n_chunks - 4) // 2, pair, 0)

        drain_gather(0)
        wait_store(0)
        transpose(0)
        fire_store(n_chunks - 2, 0)
        drain_gather(1)
        wait_store(1)
        transpose(1)
        fire_store(n_chunks - 1, 1)
        wait_store(0)
        wait_store(1)

    return gather_kernel


def kernel(token_ids, weights):
    n_tok, n_seq = token_ids.shape
    dim = weights.shape[1]
    assert dim == 64 and n_tok % _CHUNK == 0
    n_bblk = n_tok // _CHUNK
    total_chunks = n_seq * n_bblk
    assert total_chunks % _NUM_WORKERS == 0
    n_chunks = total_chunks // _NUM_WORKERS
    assert n_chunks >= 4 and n_chunks % 2 == 0
    idx = token_ids.T.reshape(_NUM_WORKERS, n_chunks, _CHUNK)
    out5 = _make_gather(n_chunks, n_seq, n_bblk)(weights, idx)
    return out5.transpose(2, 4, 0, 1, 3).reshape(n_tok, n_seq, dim)


# speedup vs baseline: 1.8566x; 1.8566x over previous
import functools

import jax
import jax.numpy as jnp
from jax import lax
from jax.experimental import pallas as pl
from jax.experimental.pallas import tpu as pltpu
from jax.experimental.pallas import tpu_sc as plsc

_NUM_CORES = 2
_NUM_SUBCORES = 16
_NUM_WORKERS = _NUM_CORES * _NUM_SUBCORES
_CHUNK = 128
_LANES = 16
_TPAD = 131


@functools.lru_cache(maxsize=None)
def _make_gather(n_chunks: int, n_seq: int, n_bblk: int):
    dim = 64
    mesh = plsc.VectorSubcoreMesh(core_axis_name="c", subcore_axis_name="s")

    @functools.partial(
        pl.kernel,
        mesh=mesh,
        out_type=jax.ShapeDtypeStruct(
            (n_seq, dim // 8, n_bblk, 8, _CHUNK), jnp.float32
        ),
        scratch_types=[
            pltpu.VMEM((n_chunks, _CHUNK), jnp.int32),
            pltpu.VMEM((_CHUNK, dim), jnp.float32),
            pltpu.VMEM((_CHUNK, dim), jnp.float32),
            pltpu.VMEM((dim // 8, 8, _TPAD), jnp.float32),
            pltpu.VMEM((dim // 8, 8, _TPAD), jnp.float32),
            pltpu.SemaphoreType.DMA,
            pltpu.SemaphoreType.DMA,
            pltpu.SemaphoreType.DMA,
            pltpu.SemaphoreType.DMA,
        ],
        compiler_params=pltpu.CompilerParams(
            use_tc_tiling_on_sc=False, needs_layout_passes=False
        ),
    )
    def gather_kernel(table_hbm, idx_hbm, out_hbm, idx_v, gb0, gb1, tb0, tb1,
                      gs0, gs1, ss0, ss1):
        wid = lax.axis_index("s") * _NUM_CORES + lax.axis_index("c")
        pltpu.sync_copy(idx_hbm.at[wid], idx_v)
        gbufs = (gb0, gb1)
        tbufs = (tb0, tb1)
        gsems = (gs0, gs1)
        ssems = (ss0, ss1)
        lanes = lax.iota(jnp.int32, _LANES)

        def fire_gather(j, p):
            pltpu.async_copy(table_hbm.at[idx_v.at[j]], gbufs[p], gsems[p])

        def drain_gather(p):
            pltpu.make_async_copy(
                table_hbm.at[pl.ds(0, _CHUNK)], gbufs[p], gsems[p]
            ).wait()

        i0b = lanes // 8
        i1 = lax.rem(lanes, 8)

        def transpose(p):
            gb = gbufs[p]
            tb = tbufs[p]

            def body(tg, carry):
                t0 = tg * _LANES
                for tt in range(_LANES):
                    t = t0 + tt
                    tcol = jnp.full((_LANES,), t, jnp.int32)
                    for dt in range(dim // _LANES):
                        v = gb[t, pl.ds(dt * _LANES, _LANES)]
                        plsc.store_scatter(tb, [dt * 2 + i0b, i1, tcol], v)
                return carry

            lax.fori_loop(0, _CHUNK // _LANES, body, 0)

        def fire_store(j, p):
            c = wid * n_chunks + j
            s = c // n_bblk
            bblk = lax.rem(c, n_bblk)
            pltpu.async_copy(
                tbufs[p].at[:, :, pl.ds(0, _CHUNK)],
                out_hbm.at[s].at[:, bblk],
                ssems[p],
            )

        def wait_store(p):
            pltpu.make_async_copy(
                tbufs[p].at[:, :, pl.ds(0, _CHUNK)],
                out_hbm.at[0].at[:, 0],
                ssems[p],
            ).wait()

        fire_gather(0, 0)
        fire_gather(1, 1)
        drain_gather(0)
        transpose(0)
        fire_store(0, 0)
        fire_gather(2, 0)
        drain_gather(1)
        transpose(1)
        fire_store(1, 1)
        fire_gather(3, 1)

        def pair(t, carry):
            j = 2 * t + 2
            drain_gather(0)
            wait_store(0)
            transpose(0)
            fire_store(j, 0)
            fire_gather(j + 2, 0)
            drain_gather(1)
            wait_store(1)
            transpose(1)
            fire_store(j + 1, 1)
            fire_gather(j + 3, 1)
            return carry

        lax.fori_loop(0, (n_chunks - 4) // 2, pair, 0)

        drain_gather(0)
        wait_store(0)
        transpose(0)
        fire_store(n_chunks - 2, 0)
        drain_gather(1)
        wait_store(1)
        transpose(1)
        fire_store(n_chunks - 1, 1)
        wait_store(0)
        wait_store(1)

    return gather_kernel


def kernel(token_ids, weights):
    n_tok, n_seq = token_ids.shape
    dim = weights.shape[1]
    assert dim == 64 and n_tok % _CHUNK == 0
    n_bblk = n_tok // _CHUNK
    total_chunks = n_seq * n_bblk
    assert total_chunks % _NUM_WORKERS == 0
    n_chunks = total_chunks // _NUM_WORKERS
    assert n_chunks >= 4 and n_chunks % 2 == 0
    idx = token_ids.T.reshape(_NUM_WORKERS, n_chunks, _CHUNK)
    out5 = _make_gather(n_chunks, n_seq, n_bblk)(weights, idx)
    return out5.transpose(2, 4, 0, 1, 3).reshape(n_tok, n_seq, dim)
